# baseline (speedup 1.0000x reference)
import functools

import jax
import jax.numpy as jnp
from jax import lax
from jax.experimental import pallas as pl
from jax.experimental.pallas import tpu as pltpu
from jax.experimental.pallas import tpu_sc as plsc

ALPHA = 0.9
ONE_MINUS_ALPHA = 0.1
NSUB = 16
NCORE = 2
H = 8
DUM = 128
KE = 256
SUB = 256
NJ = KE // SUB
NSLOT = 8
DEPTH = NSLOT // 2
ZRR = 1024


def _vrsqrt(v):
    i = plsc.bitcast(v, jnp.int32)
    h = jnp.int32(0x5F3759DF) - (i >> 1)
    r = plsc.bitcast(h, jnp.float32)
    for _ in range(3):
        r = r * (jnp.float32(1.5) - jnp.float32(0.5) * v * r * r)
    return r


def _build(N, E):
    CHUNK = ((N + NSUB - 1) // NSUB + 63) // 64 * 64
    NP = NSUB * CHUNK
    NSP = NP + DUM
    SLAB = CHUNK // 8
    EPAD = (E + NSUB * KE * NSLOT - 1) // (NSUB * KE * NSLOT) * (NSUB * KE * NSLOT)
    ET = EPAD // NSUB
    NCH = ET // KE
    ERB = ET // SUB
    A8 = NSP // NSUB

    mesh = plsc.VectorSubcoreMesh(core_axis_name="c", subcore_axis_name="s")

    @functools.partial(
        pl.kernel,
        out_type=(jax.ShapeDtypeStruct((NCORE, NP, H), jnp.float32),
                  jax.ShapeDtypeStruct((NCORE, NSP, H), jnp.float32)),
        mesh=mesh,
        scratch_types=dict(
            agg_sp=pltpu.VMEM_SHARED((NSP, H), jnp.float32),
            zer=pltpu.VMEM((ZRR, H), jnp.float32),
            dbuf=pltpu.VMEM((CHUNK,), jnp.float32),
            abuf=pltpu.VMEM((SLAB, H), jnp.float32),
            lbuf=pltpu.VMEM((SLAB, H), jnp.float32),
            isrc=pltpu.VMEM((NSLOT, NJ, SUB), jnp.int32),
            idst=pltpu.VMEM((NSLOT, NJ, SUB), jnp.int32),
            rows8=pltpu.VMEM((NSLOT, NJ, SUB, H), jnp.float32),
            onesb=pltpu.VMEM((SUB, H), jnp.float32),
            nlb=pltpu.VMEM((16,), jnp.int32),
            gsem=pltpu.SemaphoreType.DMA,
            isem=pltpu.SemaphoreType.DMA,
            ssem=pltpu.SemaphoreType.DMA,
        ),
        compiler_params=pltpu.CompilerParams(needs_layout_passes=False,
                                             use_tc_tiling_on_sc=False),
    )
    def prop(lab2d, src2d, dst2d, nl16, out_f, ys_hbm, agg_sp, zer, dbuf,
             abuf, lbuf, isrc, idst, rows8, onesb, nlb, gsem, isem, ssem):
        c = lax.axis_index("c")
        s = lax.axis_index("s")
        lane = lax.iota(jnp.int32, 16)
        hi_half = lane >> 3
        lane8 = jnp.bitwise_and(lane, 7)
        zero16 = jnp.zeros(16, jnp.int32)

        def fill_zer(k, _):
            plsc.store_scatter(zer, [2 * k + hi_half, lane8],
                               jnp.zeros(16, jnp.float32))
            return _

        lax.fori_loop(0, ZRR // 2, fill_zer, None)

        def fill_ones(k, _):
            plsc.store_scatter(onesb, [2 * k + hi_half, lane8],
                               jnp.full(16, 1.0, jnp.float32))
            return _

        lax.fori_loop(0, SUB // 2, fill_ones, None)
        pltpu.sync_copy(nl16, nlb)
        nl = jnp.max(nlb[...])

        def zero_agg(j, _):
            pltpu.sync_copy(zer, agg_sp.at[pl.ds(s * A8 + j * ZRR, ZRR)])
            return _

        nfull = A8 // ZRR
        lax.fori_loop(0, nfull, zero_agg, None)
        tail = A8 - nfull * ZRR
        if tail:
            pltpu.sync_copy(zer.at[pl.ds(0, tail)],
                            agg_sp.at[pl.ds(s * A8 + nfull * ZRR, tail)])

        @pl.when(s == 0)
        def _():
            pltpu.sync_copy(zer.at[pl.ds(0, DUM)],
                            ys_hbm.at[c, pl.ds(NP, DUM)])

        plsc.subcore_barrier()

        def edge_sweep(do_gather):
            def fire_idx(ch, b):
                rb = s * ERB + ch * NJ
                if do_gather:
                    pltpu.async_copy(src2d.at[pl.ds(rb, NJ)], isrc.at[b], isem)
                pltpu.async_copy(dst2d.at[pl.ds(rb, NJ)], idst.at[b], isem)

            def drain_scatters(b):
                for j in range(NJ):
                    src = rows8.at[b, j] if do_gather else onesb
                    pltpu.make_async_copy(src, agg_sp.at[idst.at[b, j]],
                                          ssem).wait()

            def do_chunk(ch, b):
                rb = s * ERB + ch * NJ
                if do_gather:
                    pltpu.make_async_copy(src2d.at[pl.ds(rb, NJ)], isrc.at[b],
                                          isem).wait()
                pltpu.make_async_copy(dst2d.at[pl.ds(rb, NJ)], idst.at[b],
                                      isem).wait()
                gds = []
                if do_gather:
                    gds = [
                        pltpu.async_copy(ys_hbm.at[c].at[isrc.at[b, j]],
                                         rows8.at[b, j], gsem)
                        for j in range(NJ)
                    ]

                @pl.when(ch >= DEPTH)
                def _():
                    drain_scatters((b + DEPTH) % NSLOT)

                @pl.when(ch + DEPTH < NCH)
                def _():
                    fire_idx(ch + DEPTH, (b + DEPTH) % NSLOT)

                for j in range(NJ):
                    if do_gather:
                        gds[j].wait()
                        pltpu.async_copy(rows8.at[b, j],
                                         agg_sp.at[idst.at[b, j]], ssem,
                                         add=True)
                    else:
                        pltpu.async_copy(onesb, agg_sp.at[idst.at[b, j]],
                                         ssem, add=True)

            for ch0 in range(DEPTH):
                fire_idx(ch0, ch0)

            def quad(q, _):
                for p in range(NSLOT):
                    do_chunk(q * NSLOT + p, p)
                return _

            lax.fori_loop(0, NCH // NSLOT, quad, None)
            for ch0 in range(NCH - DEPTH, NCH):
                drain_scatters(ch0 % NSLOT)

        edge_sweep(do_gather=False)
        plsc.subcore_barrier()

        def setup_slab(j, _):
            roff = s * CHUNK + j * SLAB
            pltpu.sync_copy(agg_sp.at[pl.ds(roff, SLAB)], abuf)
            pltpu.sync_copy(zer.at[pl.ds(0, SLAB)],
                            agg_sp.at[pl.ds(roff, SLAB)])
            pltpu.sync_copy(lab2d.at[c, pl.ds(roff, SLAB)], lbuf)

            def dbody(k, _):
                d = plsc.load_gather(abuf, [k * 16 + lane, zero16])
                d = jnp.maximum(d, jnp.float32(1.0))
                dbuf[pl.ds(j * SLAB + k * 16, 16)] = _vrsqrt(d)
                return _

            lax.fori_loop(0, SLAB // 16, dbody, None)

            def ybody(k, _):
                idxr = 2 * k + hi_half
                ne = plsc.load_gather(dbuf, [j * SLAB + idxr])
                l = plsc.load_gather(lbuf, [idxr, lane8])
                plsc.store_scatter(lbuf, [idxr, lane8], l * ne)
                return _

            lax.fori_loop(0, SLAB // 2, ybody, None)
            pltpu.sync_copy(lbuf, ys_hbm.at[c, pl.ds(roff, SLAB)])
            return _

        lax.fori_loop(0, 8, setup_slab, None)
        plsc.subcore_barrier()

        def iteration(it, _):
            edge_sweep(do_gather=True)
            plsc.subcore_barrier()

            def node_slab(j, _):
                roff = s * CHUNK + j * SLAB
                pltpu.sync_copy(agg_sp.at[pl.ds(roff, SLAB)], abuf)
                pltpu.sync_copy(lab2d.at[c, pl.ds(roff, SLAB)], lbuf)
                pltpu.sync_copy(zer.at[pl.ds(0, SLAB)],
                                agg_sp.at[pl.ds(roff, SLAB)])

                def body(k, _):
                    idxr = 2 * k + hi_half
                    ne = plsc.load_gather(dbuf, [j * SLAB + idxr])
                    a = plsc.load_gather(abuf, [idxr, lane8])
                    l = plsc.load_gather(lbuf, [idxr, lane8])
                    y = ONE_MINUS_ALPHA * l + ALPHA * ne * a
                    y = jnp.minimum(jnp.maximum(y, jnp.float32(0.0)),
                                    jnp.float32(1.0))
                    plsc.store_scatter(abuf, [idxr, lane8], y)
                    plsc.store_scatter(lbuf, [idxr, lane8], y * ne)
                    return _

                lax.fori_loop(0, SLAB // 2, body, None)
                pltpu.sync_copy(lbuf, ys_hbm.at[c, pl.ds(roff, SLAB)])

                @pl.when(it == nl - 1)
                def _():
                    pltpu.sync_copy(abuf, out_f.at[c, pl.ds(roff, SLAB)])

                return _

            lax.fori_loop(0, 8, node_slab, None)
            plsc.subcore_barrier()
            return _

        lax.fori_loop(0, nl, iteration, None)

    return prop, NP, EPAD


def kernel(labels, edge_index, num_layers):
    N, C = labels.shape
    E = edge_index.shape[1]
    prop, NP, EPAD = _build(N, E)
    src = edge_index[0].astype(jnp.int32)
    dst = edge_index[1].astype(jnp.int32)
    pad = NP + jnp.arange(EPAD - E, dtype=jnp.int32) % DUM
    src2d = jnp.concatenate([src, pad]).reshape(EPAD // SUB, SUB)
    dst2d = jnp.concatenate([dst, pad]).reshape(EPAD // SUB, SUB)
    labp = jnp.pad(labels, ((0, NP - N), (0, 0)))
    lab2d = jnp.stack([labp[:, :H], labp[:, H:]])
    nl16 = jnp.full((16,), num_layers, jnp.int32)
    out, _ys = prop(lab2d, src2d, dst2d, nl16)
    return jnp.concatenate([out[0, :N], out[1, :N]], axis=1)


# speedup vs baseline: 1.6171x; 1.6171x over previous
import functools

import jax
import jax.numpy as jnp
from jax import lax
from jax.experimental import pallas as pl
from jax.experimental.pallas import tpu as pltpu
from jax.experimental.pallas import tpu_sc as plsc

ALPHA = 0.9
ONE_MINUS_ALPHA = 0.1
NSUB = 16
NCORE = 2
H = 8
DUM = 128
KE = 1024
SUB = 512
NJ = KE // SUB
NSLOT = 2
DEPTH = NSLOT // 2
ZRR = 1024


def _vrsqrt(v):
    i = plsc.bitcast(v, jnp.int32)
    h = jnp.int32(0x5F3759DF) - (i >> 1)
    r = plsc.bitcast(h, jnp.float32)
    for _ in range(3):
        r = r * (jnp.float32(1.5) - jnp.float32(0.5) * v * r * r)
    return r


def _build(N, E):
    CHUNK = ((N + NSUB - 1) // NSUB + 63) // 64 * 64
    NP = NSUB * CHUNK
    NSP = NP + DUM
    SLAB = CHUNK // 8
    EPAD = (E + NSUB * KE * NSLOT - 1) // (NSUB * KE * NSLOT) * (NSUB * KE * NSLOT)
    ET = EPAD // NSUB
    NCH = ET // KE
    ERB = ET // SUB
    A8 = NSP // NSUB

    mesh = plsc.VectorSubcoreMesh(core_axis_name="c", subcore_axis_name="s")

    @functools.partial(
        pl.kernel,
        out_type=(jax.ShapeDtypeStruct((NCORE, NP, H), jnp.float32),
                  jax.ShapeDtypeStruct((NCORE, NSP, H), jnp.float32)),
        mesh=mesh,
        scratch_types=dict(
            agg_sp=pltpu.VMEM_SHARED((NSP, H), jnp.float32),
            zer=pltpu.VMEM((ZRR, H), jnp.float32),
            dbuf=pltpu.VMEM((CHUNK,), jnp.float32),
            abuf=pltpu.VMEM((SLAB, H), jnp.float32),
            lbuf=pltpu.VMEM((SLAB, H), jnp.float32),
            isrc=pltpu.VMEM((NSLOT, NJ, SUB), jnp.int32),
            idst=pltpu.VMEM((NSLOT, NJ, SUB), jnp.int32),
            rows8=pltpu.VMEM((NSLOT, NJ, SUB, H), jnp.float32),
            onesb=pltpu.VMEM((SUB, H), jnp.float32),
            nlb=pltpu.VMEM((16,), jnp.int32),
            gsem=pltpu.SemaphoreType.DMA,
            isem=pltpu.SemaphoreType.DMA,
            ssem=pltpu.SemaphoreType.DMA,
        ),
        compiler_params=pltpu.CompilerParams(needs_layout_passes=False,
                                             use_tc_tiling_on_sc=False),
    )
    def prop(lab2d, src2d, dst2d, nl16, out_f, ys_hbm, agg_sp, zer, dbuf,
             abuf, lbuf, isrc, idst, rows8, onesb, nlb, gsem, isem, ssem):
        c = lax.axis_index("c")
        s = lax.axis_index("s")
        lane = lax.iota(jnp.int32, 16)
        hi_half = lane >> 3
        lane8 = jnp.bitwise_and(lane, 7)
        zero16 = jnp.zeros(16, jnp.int32)

        def fill_zer(k, _):
            plsc.store_scatter(zer, [2 * k + hi_half, lane8],
                               jnp.zeros(16, jnp.float32))
            return _

        lax.fori_loop(0, ZRR // 2, fill_zer, None)

        def fill_ones(k, _):
            plsc.store_scatter(onesb, [2 * k + hi_half, lane8],
                               jnp.full(16, 1.0, jnp.float32))
            return _

        lax.fori_loop(0, SUB // 2, fill_ones, None)
        pltpu.sync_copy(nl16, nlb)
        nl = jnp.max(nlb[...])

        def zero_agg(j, _):
            pltpu.sync_copy(zer, agg_sp.at[pl.ds(s * A8 + j * ZRR, ZRR)])
            return _

        nfull = A8 // ZRR
        lax.fori_loop(0, nfull, zero_agg, None)
        tail = A8 - nfull * ZRR
        if tail:
            pltpu.sync_copy(zer.at[pl.ds(0, tail)],
                            agg_sp.at[pl.ds(s * A8 + nfull * ZRR, tail)])

        @pl.when(s == 0)
        def _():
            pltpu.sync_copy(zer.at[pl.ds(0, DUM)],
                            ys_hbm.at[c, pl.ds(NP, DUM)])

        plsc.subcore_barrier()

        def edge_sweep(do_gather):
            def fire_idx(ch, b):
                rb = s * ERB + ch * NJ
                if do_gather:
                    pltpu.async_copy(src2d.at[pl.ds(rb, NJ)], isrc.at[b], isem)
                pltpu.async_copy(dst2d.at[pl.ds(rb, NJ)], idst.at[b], isem)

            def drain_scatters(b):
                for j in range(NJ):
                    src = rows8.at[b, j] if do_gather else onesb
                    pltpu.make_async_copy(src, agg_sp.at[idst.at[b, j]],
                                          ssem).wait()

            def do_chunk(ch, b):
                rb = s * ERB + ch * NJ
                if do_gather:
                    pltpu.make_async_copy(src2d.at[pl.ds(rb, NJ)], isrc.at[b],
                                          isem).wait()
                pltpu.make_async_copy(dst2d.at[pl.ds(rb, NJ)], idst.at[b],
                                      isem).wait()
                gds = []
                if do_gather:
                    gds = [
                        pltpu.async_copy(ys_hbm.at[c].at[isrc.at[b, j]],
                                         rows8.at[b, j], gsem)
                        for j in range(NJ)
                    ]

                @pl.when(ch >= DEPTH)
                def _():
                    drain_scatters((b + DEPTH) % NSLOT)

                @pl.when(ch + DEPTH < NCH)
                def _():
                    fire_idx(ch + DEPTH, (b + DEPTH) % NSLOT)

                for j in range(NJ):
                    if do_gather:
                        gds[j].wait()
                        pltpu.async_copy(rows8.at[b, j],
                                         agg_sp.at[idst.at[b, j]], ssem,
                                         add=True)
                    else:
                        pltpu.async_copy(onesb, agg_sp.at[idst.at[b, j]],
                                         ssem, add=True)

            for ch0 in range(DEPTH):
                fire_idx(ch0, ch0)

            def quad(q, _):
                for p in range(NSLOT):
                    do_chunk(q * NSLOT + p, p)
                return _

            lax.fori_loop(0, NCH // NSLOT, quad, None)
            for ch0 in range(NCH - DEPTH, NCH):
                drain_scatters(ch0 % NSLOT)

        edge_sweep(do_gather=False)
        plsc.subcore_barrier()

        def setup_slab(j, _):
            roff = s * CHUNK + j * SLAB
            pltpu.sync_copy(agg_sp.at[pl.ds(roff, SLAB)], abuf)
            pltpu.sync_copy(zer.at[pl.ds(0, SLAB)],
                            agg_sp.at[pl.ds(roff, SLAB)])
            pltpu.sync_copy(lab2d.at[c, pl.ds(roff, SLAB)], lbuf)

            def dbody(k, _):
                d = plsc.load_gather(abuf, [k * 16 + lane, zero16])
                d = jnp.maximum(d, jnp.float32(1.0))
                dbuf[pl.ds(j * SLAB + k * 16, 16)] = _vrsqrt(d)
                return _

            lax.fori_loop(0, SLAB // 16, dbody, None)

            def ybody(k, _):
                idxr = 2 * k + hi_half
                ne = plsc.load_gather(dbuf, [j * SLAB + idxr])
                l = plsc.load_gather(lbuf, [idxr, lane8])
                plsc.store_scatter(lbuf, [idxr, lane8], l * ne)
                return _

            lax.fori_loop(0, SLAB // 2, ybody, None)
            pltpu.sync_copy(lbuf, ys_hbm.at[c, pl.ds(roff, SLAB)])
            return _

        lax.fori_loop(0, 8, setup_slab, None)
        plsc.subcore_barrier()

        def iteration(it, _):
            edge_sweep(do_gather=True)
            plsc.subcore_barrier()

            def node_slab(j, _):
                roff = s * CHUNK + j * SLAB
                pltpu.sync_copy(agg_sp.at[pl.ds(roff, SLAB)], abuf)
                pltpu.sync_copy(lab2d.at[c, pl.ds(roff, SLAB)], lbuf)
                pltpu.sync_copy(zer.at[pl.ds(0, SLAB)],
                                agg_sp.at[pl.ds(roff, SLAB)])

                def body(k, _):
                    idxr = 2 * k + hi_half
                    ne = plsc.load_gather(dbuf, [j * SLAB + idxr])
                    a = plsc.load_gather(abuf, [idxr, lane8])
                    l = plsc.load_gather(lbuf, [idxr, lane8])
                    y = ONE_MINUS_ALPHA * l + ALPHA * ne * a
                    y = jnp.minimum(jnp.maximum(y, jnp.float32(0.0)),
                                    jnp.float32(1.0))
                    plsc.store_scatter(abuf, [idxr, lane8], y)
                    plsc.store_scatter(lbuf, [idxr, lane8], y * ne)
                    return _

                lax.fori_loop(0, SLAB // 2, body, None)
                pltpu.sync_copy(lbuf, ys_hbm.at[c, pl.ds(roff, SLAB)])

                @pl.when(it == nl - 1)
                def _():
                    pltpu.sync_copy(abuf, out_f.at[c, pl.ds(roff, SLAB)])

                return _

            lax.fori_loop(0, 8, node_slab, None)
            plsc.subcore_barrier()
            return _

        lax.fori_loop(0, nl, iteration, None)

    return prop, NP, EPAD


def kernel(labels, edge_index, num_layers):
    N, C = labels.shape
    E = edge_index.shape[1]
    prop, NP, EPAD = _build(N, E)
    src = edge_index[0].astype(jnp.int32)
    dst = edge_index[1].astype(jnp.int32)
    pad = NP + jnp.arange(EPAD - E, dtype=jnp.int32) % DUM
    src2d = jnp.concatenate([src, pad]).reshape(EPAD // SUB, SUB)
    dst2d = jnp.concatenate([dst, pad]).reshape(EPAD // SUB, SUB)
    labp = jnp.pad(labels, ((0, NP - N), (0, 0)))
    lab2d = jnp.stack([labp[:, :H], labp[:, H:]])
    nl16 = jnp.full((16,), num_layers, jnp.int32)
    out, _ys = prop(lab2d, src2d, dst2d, nl16)
    return jnp.concatenate([out[0, :N], out[1, :N]], axis=1)


# speedup vs baseline: 1.6190x; 1.0012x over previous
import functools

import jax
import jax.numpy as jnp
from jax import lax
from jax.experimental import pallas as pl
from jax.experimental.pallas import tpu as pltpu
from jax.experimental.pallas import tpu_sc as plsc

ALPHA = 0.9
ONE_MINUS_ALPHA = 0.1
NSUB = 16
NCORE = 2
H = 8
DUM = 128
KE = 1024
SUB = 1024
NJ = KE // SUB
NSLOT = 2
DEPTH = NSLOT // 2
ZRR = 1024


def _vrsqrt(v):
    i = plsc.bitcast(v, jnp.int32)
    h = jnp.int32(0x5F3759DF) - (i >> 1)
    r = plsc.bitcast(h, jnp.float32)
    for _ in range(3):
        r = r * (jnp.float32(1.5) - jnp.float32(0.5) * v * r * r)
    return r


def _build(N, E):
    CHUNK = ((N + NSUB - 1) // NSUB + 63) // 64 * 64
    NP = NSUB * CHUNK
    NSP = NP + DUM
    SLAB = CHUNK // 8
    EPAD = (E + NSUB * KE * NSLOT - 1) // (NSUB * KE * NSLOT) * (NSUB * KE * NSLOT)
    ET = EPAD // NSUB
    NCH = ET // KE
    ERB = ET // SUB
    A8 = NSP // NSUB

    mesh = plsc.VectorSubcoreMesh(core_axis_name="c", subcore_axis_name="s")

    @functools.partial(
        pl.kernel,
        out_type=(jax.ShapeDtypeStruct((NCORE, NP, H), jnp.float32),
                  jax.ShapeDtypeStruct((NCORE, NSP, H), jnp.float32)),
        mesh=mesh,
        scratch_types=dict(
            agg_sp=pltpu.VMEM_SHARED((NSP, H), jnp.float32),
            zer=pltpu.VMEM((ZRR, H), jnp.float32),
            dbuf=pltpu.VMEM((CHUNK,), jnp.float32),
            abuf=pltpu.VMEM((SLAB, H), jnp.float32),
            lbuf=pltpu.VMEM((SLAB, H), jnp.float32),
            isrc=pltpu.VMEM((NSLOT, NJ, SUB), jnp.int32),
            idst=pltpu.VMEM((NSLOT, NJ, SUB), jnp.int32),
            rows8=pltpu.VMEM((NSLOT, NJ, SUB, H), jnp.float32),
            onesb=pltpu.VMEM((SUB, H), jnp.float32),
            nlb=pltpu.VMEM((16,), jnp.int32),
            gsem=pltpu.SemaphoreType.DMA,
            isem=pltpu.SemaphoreType.DMA,
            ssem=pltpu.SemaphoreType.DMA,
        ),
        compiler_params=pltpu.CompilerParams(needs_layout_passes=False,
                                             use_tc_tiling_on_sc=False),
    )
    def prop(lab2d, src2d, dst2d, nl16, out_f, ys_hbm, agg_sp, zer, dbuf,
             abuf, lbuf, isrc, idst, rows8, onesb, nlb, gsem, isem, ssem):
        c = lax.axis_index("c")
        s = lax.axis_index("s")
        lane = lax.iota(jnp.int32, 16)
        hi_half = lane >> 3
        lane8 = jnp.bitwise_and(lane, 7)
        zero16 = jnp.zeros(16, jnp.int32)

        def fill_zer(k, _):
            plsc.store_scatter(zer, [2 * k + hi_half, lane8],
                               jnp.zeros(16, jnp.float32))
            return _

        lax.fori_loop(0, ZRR // 2, fill_zer, None)

        def fill_ones(k, _):
            plsc.store_scatter(onesb, [2 * k + hi_half, lane8],
                               jnp.full(16, 1.0, jnp.float32))
            return _

        lax.fori_loop(0, SUB // 2, fill_ones, None)
        pltpu.sync_copy(nl16, nlb)
        nl = jnp.max(nlb[...])

        def zero_agg(j, _):
            pltpu.sync_copy(zer, agg_sp.at[pl.ds(s * A8 + j * ZRR, ZRR)])
            return _

        nfull = A8 // ZRR
        lax.fori_loop(0, nfull, zero_agg, None)
        tail = A8 - nfull * ZRR
        if tail:
            pltpu.sync_copy(zer.at[pl.ds(0, tail)],
                            agg_sp.at[pl.ds(s * A8 + nfull * ZRR, tail)])

        @pl.when(s == 0)
        def _():
            pltpu.sync_copy(zer.at[pl.ds(0, DUM)],
                            ys_hbm.at[c, pl.ds(NP, DUM)])

        plsc.subcore_barrier()

        def edge_sweep(do_gather):
            def fire_idx(ch, b):
                rb = s * ERB + ch * NJ
                if do_gather:
                    pltpu.async_copy(src2d.at[pl.ds(rb, NJ)], isrc.at[b], isem)
                pltpu.async_copy(dst2d.at[pl.ds(rb, NJ)], idst.at[b], isem)

            def drain_scatters(b):
                for j in range(NJ):
                    src = rows8.at[b, j] if do_gather else onesb
                    pltpu.make_async_copy(src, agg_sp.at[idst.at[b, j]],
                                          ssem).wait()

            def do_chunk(ch, b):
                rb = s * ERB + ch * NJ
                if do_gather:
                    pltpu.make_async_copy(src2d.at[pl.ds(rb, NJ)], isrc.at[b],
                                          isem).wait()
                pltpu.make_async_copy(dst2d.at[pl.ds(rb, NJ)], idst.at[b],
                                      isem).wait()
                gds = []
                if do_gather:
                    gds = [
                        pltpu.async_copy(ys_hbm.at[c].at[isrc.at[b, j]],
                                         rows8.at[b, j], gsem)
                        for j in range(NJ)
                    ]

                @pl.when(ch >= DEPTH)
                def _():
                    drain_scatters((b + DEPTH) % NSLOT)

                @pl.when(ch + DEPTH < NCH)
                def _():
                    fire_idx(ch + DEPTH, (b + DEPTH) % NSLOT)

                for j in range(NJ):
                    if do_gather:
                        gds[j].wait()
                        pltpu.async_copy(rows8.at[b, j],
                                         agg_sp.at[idst.at[b, j]], ssem,
                                         add=True)
                    else:
                        pltpu.async_copy(onesb, agg_sp.at[idst.at[b, j]],
                                         ssem, add=True)

            for ch0 in range(DEPTH):
                fire_idx(ch0, ch0)

            def quad(q, _):
                for p in range(NSLOT):
                    do_chunk(q * NSLOT + p, p)
                return _

            lax.fori_loop(0, NCH // NSLOT, quad, None)
            for ch0 in range(NCH - DEPTH, NCH):
                drain_scatters(ch0 % NSLOT)

        edge_sweep(do_gather=False)
        plsc.subcore_barrier()

        def setup_slab(j, _):
            roff = s * CHUNK + j * SLAB
            pltpu.sync_copy(agg_sp.at[pl.ds(roff, SLAB)], abuf)
            pltpu.sync_copy(zer.at[pl.ds(0, SLAB)],
                            agg_sp.at[pl.ds(roff, SLAB)])
            pltpu.sync_copy(lab2d.at[c, pl.ds(roff, SLAB)], lbuf)

            def dbody(k, _):
                d = plsc.load_gather(abuf, [k * 16 + lane, zero16])
                d = jnp.maximum(d, jnp.float32(1.0))
                dbuf[pl.ds(j * SLAB + k * 16, 16)] = _vrsqrt(d)
                return _

            lax.fori_loop(0, SLAB // 16, dbody, None)

            def ybody(k, _):
                idxr = 2 * k + hi_half
                ne = plsc.load_gather(dbuf, [j * SLAB + idxr])
                l = plsc.load_gather(lbuf, [idxr, lane8])
                plsc.store_scatter(lbuf, [idxr, lane8], l * ne)
                return _

            lax.fori_loop(0, SLAB // 2, ybody, None)
            pltpu.sync_copy(lbuf, ys_hbm.at[c, pl.ds(roff, SLAB)])
            return _

        lax.fori_loop(0, 8, setup_slab, None)
        plsc.subcore_barrier()

        def iteration(it, _):
            edge_sweep(do_gather=True)
            plsc.subcore_barrier()

            def node_slab(j, _):
                roff = s * CHUNK + j * SLAB
                pltpu.sync_copy(agg_sp.at[pl.ds(roff, SLAB)], abuf)
                pltpu.sync_copy(lab2d.at[c, pl.ds(roff, SLAB)], lbuf)
                pltpu.sync_copy(zer.at[pl.ds(0, SLAB)],
                                agg_sp.at[pl.ds(roff, SLAB)])

                def body(k, _):
                    idxr = 2 * k + hi_half
                    ne = plsc.load_gather(dbuf, [j * SLAB + idxr])
                    a = plsc.load_gather(abuf, [idxr, lane8])
                    l = plsc.load_gather(lbuf, [idxr, lane8])
                    y = ONE_MINUS_ALPHA * l + ALPHA * ne * a
                    y = jnp.minimum(jnp.maximum(y, jnp.float32(0.0)),
                                    jnp.float32(1.0))
                    plsc.store_scatter(abuf, [idxr, lane8], y)
                    plsc.store_scatter(lbuf, [idxr, lane8], y * ne)
                    return _

                lax.fori_loop(0, SLAB // 2, body, None)
                pltpu.sync_copy(lbuf, ys_hbm.at[c, pl.ds(roff, SLAB)])

                @pl.when(it == nl - 1)
                def _():
                    pltpu.sync_copy(abuf, out_f.at[c, pl.ds(roff, SLAB)])

                return _

            lax.fori_loop(0, 8, node_slab, None)
            plsc.subcore_barrier()
            return _

        lax.fori_loop(0, nl, iteration, None)

    return prop, NP, EPAD


def kernel(labels, edge_index, num_layers):
    N, C = labels.shape
    E = edge_index.shape[1]
    prop, NP, EPAD = _build(N, E)
    src = edge_index[0].astype(jnp.int32)
    dst = edge_index[1].astype(jnp.int32)
    pad = NP + jnp.arange(EPAD - E, dtype=jnp.int32) % DUM
    src2d = jnp.concatenate([src, pad]).reshape(EPAD // SUB, SUB)
    dst2d = jnp.concatenate([dst, pad]).reshape(EPAD // SUB, SUB)
    labp = jnp.pad(labels, ((0, NP - N), (0, 0)))
    lab2d = jnp.stack([labp[:, :H], labp[:, H:]])
    nl16 = jnp.full((16,), num_layers, jnp.int32)
    out, _ys = prop(lab2d, src2d, dst2d, nl16)
    return jnp.concatenate([out[0, :N], out[1, :N]], axis=1)


# speedup vs baseline: 1.8218x; 1.1253x over previous
import functools

import jax
import jax.numpy as jnp
from jax import lax
from jax.experimental import pallas as pl
from jax.experimental.pallas import tpu as pltpu
from jax.experimental.pallas import tpu_sc as plsc

ALPHA = 0.9
ONE_MINUS_ALPHA = 0.1
NSUB = 16
NCORE = 2
H = 8
DUM = 128
KE = 2048
SUB = 1024
NJ = KE // SUB
NSLOT = 2
DEPTH = NSLOT // 2
ZRR = 1024


def _vrsqrt(v):
    i = plsc.bitcast(v, jnp.int32)
    h = jnp.int32(0x5F3759DF) - (i >> 1)
    r = plsc.bitcast(h, jnp.float32)
    for _ in range(3):
        r = r * (jnp.float32(1.5) - jnp.float32(0.5) * v * r * r)
    return r


def _build(N, E):
    CHUNK = ((N + NSUB - 1) // NSUB + 63) // 64 * 64
    NP = NSUB * CHUNK
    NSP = NP + DUM
    SLAB = CHUNK // 8
    EPAD = (E + NSUB * KE * NSLOT - 1) // (NSUB * KE * NSLOT) * (NSUB * KE * NSLOT)
    ET = EPAD // NSUB
    NCH = ET // KE
    ERB = ET // SUB
    A8 = NSP // NSUB

    mesh = plsc.VectorSubcoreMesh(core_axis_name="c", subcore_axis_name="s")

    @functools.partial(
        pl.kernel,
        out_type=(jax.ShapeDtypeStruct((NCORE, NP, H), jnp.float32),
                  jax.ShapeDtypeStruct((NCORE, NSP, H), jnp.float32)),
        mesh=mesh,
        scratch_types=dict(
            agg_sp=pltpu.VMEM_SHARED((NSP, H), jnp.float32),
            zer=pltpu.VMEM((ZRR, H), jnp.float32),
            dbuf=pltpu.VMEM((CHUNK,), jnp.float32),
            abuf=pltpu.VMEM((SLAB, H), jnp.float32),
            lbuf=pltpu.VMEM((SLAB, H), jnp.float32),
            isrc=pltpu.VMEM((NSLOT, NJ, SUB), jnp.int32),
            idst=pltpu.VMEM((NSLOT, NJ, SUB), jnp.int32),
            rows8=pltpu.VMEM((NSLOT, NJ, SUB, H), jnp.float32),
            onesb=pltpu.VMEM((SUB, H), jnp.float32),
            nlb=pltpu.VMEM((16,), jnp.int32),
            gsem=pltpu.SemaphoreType.DMA,
            isem=pltpu.SemaphoreType.DMA,
            ssem=pltpu.SemaphoreType.DMA,
        ),
        compiler_params=pltpu.CompilerParams(needs_layout_passes=False,
                                             use_tc_tiling_on_sc=False),
    )
    def prop(lab2d, src2d, dst2d, nl16, out_f, ys_hbm, agg_sp, zer, dbuf,
             abuf, lbuf, isrc, idst, rows8, onesb, nlb, gsem, isem, ssem):
        c = lax.axis_index("c")
        s = lax.axis_index("s")
        lane = lax.iota(jnp.int32, 16)
        hi_half = lane >> 3
        lane8 = jnp.bitwise_and(lane, 7)
        zero16 = jnp.zeros(16, jnp.int32)

        def fill_zer(k, _):
            plsc.store_scatter(zer, [2 * k + hi_half, lane8],
                               jnp.zeros(16, jnp.float32))
            return _

        lax.fori_loop(0, ZRR // 2, fill_zer, None)

        def fill_ones(k, _):
            plsc.store_scatter(onesb, [2 * k + hi_half, lane8],
                               jnp.full(16, 1.0, jnp.float32))
            return _

        lax.fori_loop(0, SUB // 2, fill_ones, None)
        pltpu.sync_copy(nl16, nlb)
        nl = jnp.max(nlb[...])

        def zero_agg(j, _):
            pltpu.sync_copy(zer, agg_sp.at[pl.ds(s * A8 + j * ZRR, ZRR)])
            return _

        nfull = A8 // ZRR
        lax.fori_loop(0, nfull, zero_agg, None)
        tail = A8 - nfull * ZRR
        if tail:
            pltpu.sync_copy(zer.at[pl.ds(0, tail)],
                            agg_sp.at[pl.ds(s * A8 + nfull * ZRR, tail)])

        @pl.when(s == 0)
        def _():
            pltpu.sync_copy(zer.at[pl.ds(0, DUM)],
                            ys_hbm.at[c, pl.ds(NP, DUM)])

        plsc.subcore_barrier()

        def edge_sweep(do_gather):
            def fire_idx(ch, b):
                rb = s * ERB + ch * NJ
                if do_gather:
                    pltpu.async_copy(src2d.at[pl.ds(rb, NJ)], isrc.at[b], isem)
                pltpu.async_copy(dst2d.at[pl.ds(rb, NJ)], idst.at[b], isem)

            def drain_scatters(b):
                for j in range(NJ):
                    src = rows8.at[b, j] if do_gather else onesb
                    pltpu.make_async_copy(src, agg_sp.at[idst.at[b, j]],
                                          ssem).wait()

            def do_chunk(ch, b):
                rb = s * ERB + ch * NJ
                if do_gather:
                    pltpu.make_async_copy(src2d.at[pl.ds(rb, NJ)], isrc.at[b],
                                          isem).wait()
                pltpu.make_async_copy(dst2d.at[pl.ds(rb, NJ)], idst.at[b],
                                      isem).wait()
                gds = []
                if do_gather:
                    gds = [
                        pltpu.async_copy(ys_hbm.at[c].at[isrc.at[b, j]],
                                         rows8.at[b, j], gsem)
                        for j in range(NJ)
                    ]

                @pl.when(ch >= DEPTH)
                def _():
                    drain_scatters((b + DEPTH) % NSLOT)

                @pl.when(ch + DEPTH < NCH)
                def _():
                    fire_idx(ch + DEPTH, (b + DEPTH) % NSLOT)

                for j in range(NJ):
                    if do_gather:
                        gds[j].wait()
                        pltpu.async_copy(rows8.at[b, j],
                                         agg_sp.at[idst.at[b, j]], ssem,
                                         add=True)
                    else:
                        pltpu.async_copy(onesb, agg_sp.at[idst.at[b, j]],
                                         ssem, add=True)

            for ch0 in range(DEPTH):
                fire_idx(ch0, ch0)

            def quad(q, _):
                for p in range(NSLOT):
                    do_chunk(q * NSLOT + p, p)
                return _

            lax.fori_loop(0, NCH // NSLOT, quad, None)
            for ch0 in range(NCH - DEPTH, NCH):
                drain_scatters(ch0 % NSLOT)

        edge_sweep(do_gather=False)
        plsc.subcore_barrier()

        def setup_slab(j, _):
            roff = s * CHUNK + j * SLAB
            pltpu.sync_copy(agg_sp.at[pl.ds(roff, SLAB)], abuf)
            pltpu.sync_copy(zer.at[pl.ds(0, SLAB)],
                            agg_sp.at[pl.ds(roff, SLAB)])
            pltpu.sync_copy(lab2d.at[c, pl.ds(roff, SLAB)], lbuf)

            def dbody(k, _):
                d = plsc.load_gather(abuf, [k * 16 + lane, zero16])
                d = jnp.maximum(d, jnp.float32(1.0))
                dbuf[pl.ds(j * SLAB + k * 16, 16)] = _vrsqrt(d)
                return _

            lax.fori_loop(0, SLAB // 16, dbody, None)

            def ybody(k, _):
                idxr = 2 * k + hi_half
                ne = plsc.load_gather(dbuf, [j * SLAB + idxr])
                l = plsc.load_gather(lbuf, [idxr, lane8])
                plsc.store_scatter(lbuf, [idxr, lane8], l * ne)
                return _

            lax.fori_loop(0, SLAB // 2, ybody, None)
            pltpu.sync_copy(lbuf, ys_hbm.at[c, pl.ds(roff, SLAB)])
            return _

        lax.fori_loop(0, 8, setup_slab, None)
        plsc.subcore_barrier()

        def iteration(it, _):
            edge_sweep(do_gather=True)
            plsc.subcore_barrier()

            def node_slab(j, _):
                roff = s * CHUNK + j * SLAB
                pltpu.sync_copy(agg_sp.at[pl.ds(roff, SLAB)], abuf)
                pltpu.sync_copy(lab2d.at[c, pl.ds(roff, SLAB)], lbuf)
                pltpu.sync_copy(zer.at[pl.ds(0, SLAB)],
                                agg_sp.at[pl.ds(roff, SLAB)])

                def body(k, _):
                    idxr = 2 * k + hi_half
                    ne = plsc.load_gather(dbuf, [j * SLAB + idxr])
                    a = plsc.load_gather(abuf, [idxr, lane8])
                    l = plsc.load_gather(lbuf, [idxr, lane8])
                    y = ONE_MINUS_ALPHA * l + ALPHA * ne * a
                    y = jnp.minimum(jnp.maximum(y, jnp.float32(0.0)),
                                    jnp.float32(1.0))
                    plsc.store_scatter(abuf, [idxr, lane8], y)
                    plsc.store_scatter(lbuf, [idxr, lane8], y * ne)
                    return _

                lax.fori_loop(0, SLAB // 2, body, None)
                pltpu.sync_copy(lbuf, ys_hbm.at[c, pl.ds(roff, SLAB)])

                @pl.when(it == nl - 1)
                def _():
                    pltpu.sync_copy(abuf, out_f.at[c, pl.ds(roff, SLAB)])

                return _

            lax.fori_loop(0, 8, node_slab, None)
            plsc.subcore_barrier()
            return _

        lax.fori_loop(0, nl, iteration, None)

    return prop, NP, EPAD


def kernel(labels, edge_index, num_layers):
    N, C = labels.shape
    E = edge_index.shape[1]
    prop, NP, EPAD = _build(N, E)
    src = edge_index[0].astype(jnp.int32)
    dst = edge_index[1].astype(jnp.int32)
    pad = NP + jnp.arange(EPAD - E, dtype=jnp.int32) % DUM
    src2d = jnp.concatenate([src, pad]).reshape(EPAD // SUB, SUB)
    dst2d = jnp.concatenate([dst, pad]).reshape(EPAD // SUB, SUB)
    labp = jnp.pad(labels, ((0, NP - N), (0, 0)))
    lab2d = jnp.stack([labp[:, :H], labp[:, H:]])
    nl16 = jnp.full((16,), num_layers, jnp.int32)
    out, _ys = prop(lab2d, src2d, dst2d, nl16)
    return jnp.concatenate([out[0, :N], out[1, :N]], axis=1)


# speedup vs baseline: 2.0430x; 1.1214x over previous
import functools

import jax
import jax.numpy as jnp
from jax import lax
from jax.experimental import pallas as pl
from jax.experimental.pallas import tpu as pltpu
from jax.experimental.pallas import tpu_sc as plsc

ALPHA = 0.9
ONE_MINUS_ALPHA = 0.1
NSUB = 16
NCORE = 2
H = 8
DUM = 128
KE = 512
SUB = 512
NJ = KE // SUB
NSLOT = 2
DEPTH = NSLOT // 2
NSLAB = 16


def _vrsqrt(v):
    i = plsc.bitcast(v, jnp.int32)
    h = jnp.int32(0x5F3759DF) - (i >> 1)
    r = plsc.bitcast(h, jnp.float32)
    for _ in range(3):
        r = r * (jnp.float32(1.5) - jnp.float32(0.5) * v * r * r)
    return r


def _build(N, E):
    CHUNK = ((N + NSUB - 1) // NSUB + 63) // 64 * 64
    NP = NSUB * CHUNK
    NSP = NP + DUM
    SLAB = CHUNK // NSLAB
    EPAD = (E + NSUB * KE * NSLOT - 1) // (NSUB * KE * NSLOT) * (NSUB * KE * NSLOT)
    ET = EPAD // NSUB
    NCH = ET // KE
    ERB = ET // SUB
    A8 = NSP // NSUB

    mesh = plsc.VectorSubcoreMesh(core_axis_name="c", subcore_axis_name="s")

    @functools.partial(
        pl.kernel,
        out_type=jax.ShapeDtypeStruct((NCORE, NP, H), jnp.float32),
        mesh=mesh,
        scratch_types=dict(
            ys_sp=pltpu.VMEM_SHARED((NSP, H), jnp.float32),
            agg_sp=pltpu.VMEM_SHARED((NSP, H), jnp.float32),
            zer=pltpu.VMEM((SLAB, H), jnp.float32),
            dbuf=pltpu.VMEM((CHUNK + 16,), jnp.float32),
            abuf=pltpu.VMEM((SLAB, H), jnp.float32),
            lbuf=pltpu.VMEM((SLAB, H), jnp.float32),
            isrc=pltpu.VMEM((NSLOT, NJ, SUB), jnp.int32),
            idst=pltpu.VMEM((NSLOT, NJ, SUB), jnp.int32),
            rows8=pltpu.VMEM((NSLOT, NJ, SUB, H), jnp.float32),
            nlb=pltpu.VMEM((16,), jnp.int32),
            gsem=pltpu.SemaphoreType.DMA,
            isem=pltpu.SemaphoreType.DMA,
            ssem=pltpu.SemaphoreType.DMA,
        ),
        compiler_params=pltpu.CompilerParams(needs_layout_passes=False,
                                             use_tc_tiling_on_sc=False),
    )
    def prop(lab2d, src2d, dst2d, nl16, out_f, ys_sp, agg_sp, zer, dbuf,
             abuf, lbuf, isrc, idst, rows8, nlb, gsem, isem, ssem):
        c = lax.axis_index("c")
        s = lax.axis_index("s")
        lane = lax.iota(jnp.int32, 16)
        hi_half = lane >> 3
        lane8 = jnp.bitwise_and(lane, 7)
        zero16 = jnp.zeros(16, jnp.int32)

        def fill_zer(k, _):
            plsc.store_scatter(zer, [2 * k + hi_half, lane8],
                               jnp.zeros(16, jnp.float32))
            return _

        lax.fori_loop(0, SLAB // 2, fill_zer, None)
        ones0 = rows8.at[0, 0]

        def fill_ones(k, _):
            plsc.store_scatter(ones0, [2 * k + hi_half, lane8],
                               jnp.full(16, 1.0, jnp.float32))
            return _

        lax.fori_loop(0, SUB // 2, fill_ones, None)
        pltpu.sync_copy(nl16, nlb)
        nl = jnp.max(nlb[...])

        def zero_agg(j, _):
            pltpu.sync_copy(zer, agg_sp.at[pl.ds(s * A8 + j * SLAB, SLAB)])
            return _

        nfull = A8 // SLAB
        lax.fori_loop(0, nfull, zero_agg, None)
        tail = A8 - nfull * SLAB
        if tail:
            pltpu.sync_copy(zer.at[pl.ds(0, tail)],
                            agg_sp.at[pl.ds(s * A8 + nfull * SLAB, tail)])

        @pl.when(s == 0)
        def _():
            pltpu.sync_copy(zer.at[pl.ds(0, DUM)], ys_sp.at[pl.ds(NP, DUM)])

        plsc.subcore_barrier()

        def edge_sweep(do_gather):
            def fire_idx(ch, b):
                rb = s * ERB + ch * NJ
                if do_gather:
                    pltpu.async_copy(src2d.at[pl.ds(rb, NJ)], isrc.at[b], isem)
                pltpu.async_copy(dst2d.at[pl.ds(rb, NJ)], idst.at[b], isem)

            def drain_scatters(b):
                for j in range(NJ):
                    src = rows8.at[b, j] if do_gather else ones0
                    pltpu.make_async_copy(src, agg_sp.at[idst.at[b, j]],
                                          ssem).wait()

            def do_chunk(ch, b):
                rb = s * ERB + ch * NJ
                if do_gather:
                    pltpu.make_async_copy(src2d.at[pl.ds(rb, NJ)], isrc.at[b],
                                          isem).wait()
                pltpu.make_async_copy(dst2d.at[pl.ds(rb, NJ)], idst.at[b],
                                      isem).wait()
                gds = []
                if do_gather:
                    gds = [
                        pltpu.async_copy(ys_sp.at[isrc.at[b, j]],
                                         rows8.at[b, j], gsem)
                        for j in range(NJ)
                    ]

                @pl.when(ch >= DEPTH)
                def _():
                    drain_scatters((b + DEPTH) % NSLOT)

                @pl.when(ch + DEPTH < NCH)
                def _():
                    fire_idx(ch + DEPTH, (b + DEPTH) % NSLOT)

                for j in range(NJ):
                    if do_gather:
                        gds[j].wait()
                        pltpu.async_copy(rows8.at[b, j],
                                         agg_sp.at[idst.at[b, j]], ssem,
                                         add=True)
                    else:
                        pltpu.async_copy(ones0, agg_sp.at[idst.at[b, j]],
                                         ssem, add=True)

            for ch0 in range(DEPTH):
                fire_idx(ch0, ch0)

            def quad(q, _):
                for p in range(NSLOT):
                    do_chunk(q * NSLOT + p, p)
                return _

            lax.fori_loop(0, NCH // NSLOT, quad, None)
            for ch0 in range(NCH - DEPTH, NCH):
                drain_scatters(ch0 % NSLOT)

        edge_sweep(do_gather=False)
        plsc.subcore_barrier()

        def setup_slab(j, _):
            roff = s * CHUNK + j * SLAB
            pltpu.sync_copy(agg_sp.at[pl.ds(roff, SLAB)], abuf)
            pltpu.sync_copy(zer.at[pl.ds(0, SLAB)],
                            agg_sp.at[pl.ds(roff, SLAB)])
            pltpu.sync_copy(lab2d.at[c, pl.ds(roff, SLAB)], lbuf)

            def dbody(k, _):
                d = plsc.load_gather(abuf, [k * 16 + lane, zero16])
                d = jnp.maximum(d, jnp.float32(1.0))
                dbuf[pl.ds(j * SLAB + k * 16, 16)] = _vrsqrt(d)
                return _

            lax.fori_loop(0, (SLAB + 15) // 16, dbody, None)

            def ybody(k, _):
                idxr = 2 * k + hi_half
                ne = plsc.load_gather(dbuf, [j * SLAB + idxr])
                l = plsc.load_gather(lbuf, [idxr, lane8])
                plsc.store_scatter(lbuf, [idxr, lane8], l * ne)
                return _

            lax.fori_loop(0, SLAB // 2, ybody, None)
            pltpu.sync_copy(lbuf, ys_sp.at[pl.ds(roff, SLAB)])
            return _

        lax.fori_loop(0, NSLAB, setup_slab, None)
        plsc.subcore_barrier()

        def iteration(it, _):
            edge_sweep(do_gather=True)
            plsc.subcore_barrier()

            def node_slab(j, _):
                roff = s * CHUNK + j * SLAB
                pltpu.sync_copy(agg_sp.at[pl.ds(roff, SLAB)], abuf)
                pltpu.sync_copy(lab2d.at[c, pl.ds(roff, SLAB)], lbuf)
                pltpu.sync_copy(zer.at[pl.ds(0, SLAB)],
                                agg_sp.at[pl.ds(roff, SLAB)])

                def body(k, _):
                    idxr = 2 * k + hi_half
                    ne = plsc.load_gather(dbuf, [j * SLAB + idxr])
                    a = plsc.load_gather(abuf, [idxr, lane8])
                    l = plsc.load_gather(lbuf, [idxr, lane8])
                    y = ONE_MINUS_ALPHA * l + ALPHA * ne * a
                    y = jnp.minimum(jnp.maximum(y, jnp.float32(0.0)),
                                    jnp.float32(1.0))
                    plsc.store_scatter(abuf, [idxr, lane8], y)
                    plsc.store_scatter(lbuf, [idxr, lane8], y * ne)
                    return _

                lax.fori_loop(0, SLAB // 2, body, None)
                pltpu.sync_copy(lbuf, ys_sp.at[pl.ds(roff, SLAB)])

                @pl.when(it == nl - 1)
                def _():
                    pltpu.sync_copy(abuf, out_f.at[c, pl.ds(roff, SLAB)])

                return _

            lax.fori_loop(0, NSLAB, node_slab, None)
            plsc.subcore_barrier()
            return _

        lax.fori_loop(0, nl, iteration, None)

    return prop, NP, EPAD


def kernel(labels, edge_index, num_layers):
    N, C = labels.shape
    E = edge_index.shape[1]
    prop, NP, EPAD = _build(N, E)
    src = edge_index[0].astype(jnp.int32)
    dst = edge_index[1].astype(jnp.int32)
    pad = NP + jnp.arange(EPAD - E, dtype=jnp.int32) % DUM
    src2d = jnp.concatenate([src, pad]).reshape(EPAD // SUB, SUB)
    dst2d = jnp.concatenate([dst, pad]).reshape(EPAD // SUB, SUB)
    labp = jnp.pad(labels, ((0, NP - N), (0, 0)))
    lab2d = jnp.stack([labp[:, :H], labp[:, H:]])
    nl16 = jnp.full((16,), num_layers, jnp.int32)
    out = prop(lab2d, src2d, dst2d, nl16)
    return jnp.concatenate([out[0, :N], out[1, :N]], axis=1)


# speedup vs baseline: 2.1479x; 1.0513x over previous
import functools

import jax
import jax.numpy as jnp
from jax import lax
from jax.experimental import pallas as pl
from jax.experimental.pallas import tpu as pltpu
from jax.experimental.pallas import tpu_sc as plsc

ALPHA = 0.9
ONE_MINUS_ALPHA = 0.1
NSUB = 16
NCORE = 2
H = 8
DUM = 128
KE = 512
SUB = 512
NJ = KE // SUB
NSLOT = 2
DEPTH = NSLOT // 2
NSLAB = 16


def _vrsqrt(v):
    i = plsc.bitcast(v, jnp.int32)
    h = jnp.int32(0x5F3759DF) - (i >> 1)
    r = plsc.bitcast(h, jnp.float32)
    for _ in range(3):
        r = r * (jnp.float32(1.5) - jnp.float32(0.5) * v * r * r)
    return r


def _build(N, E):
    CHUNK = ((N + NSUB - 1) // NSUB + 63) // 64 * 64
    NP = NSUB * CHUNK
    NSP = NP + DUM
    SLAB = CHUNK // NSLAB
    EPAD = (E + NSUB * KE * NSLOT - 1) // (NSUB * KE * NSLOT) * (NSUB * KE * NSLOT)
    ET = EPAD // NSUB
    NCH = ET // KE
    ERB = ET // SUB
    A8 = NSP // NSUB

    mesh = plsc.VectorSubcoreMesh(core_axis_name="c", subcore_axis_name="s")

    @functools.partial(
        pl.kernel,
        out_type=jax.ShapeDtypeStruct((NCORE, NP, H), jnp.float32),
        mesh=mesh,
        scratch_types=dict(
            ys_sp=pltpu.VMEM_SHARED((NSP, H), jnp.float32),
            agg_sp=pltpu.VMEM_SHARED((NSP, H), jnp.float32),
            zer=pltpu.VMEM((SLAB, H), jnp.float32),
            dbuf=pltpu.VMEM((CHUNK + 16,), jnp.float32),
            abuf=pltpu.VMEM((SLAB, H), jnp.float32),
            lbuf=pltpu.VMEM((2, SLAB, H), jnp.float32),
            isrc=pltpu.VMEM((NSLOT, NJ, SUB), jnp.int32),
            idst=pltpu.VMEM((NSLOT, NJ, SUB), jnp.int32),
            rows8=pltpu.VMEM((NSLOT, NJ, SUB, H), jnp.float32),
            nlb=pltpu.VMEM((16,), jnp.int32),
            gsem=pltpu.SemaphoreType.DMA,
            lsem=pltpu.SemaphoreType.DMA,
            isem=pltpu.SemaphoreType.DMA,
            ssem=pltpu.SemaphoreType.DMA,
        ),
        compiler_params=pltpu.CompilerParams(needs_layout_passes=False,
                                             use_tc_tiling_on_sc=False),
    )
    def prop(lab2d, src2d, dst2d, nl16, out_f, ys_sp, agg_sp, zer, dbuf,
             abuf, lbuf, isrc, idst, rows8, nlb, gsem, lsem, isem, ssem):
        c = lax.axis_index("c")
        s = lax.axis_index("s")
        lane = lax.iota(jnp.int32, 16)
        hi_half = lane >> 3
        lane8 = jnp.bitwise_and(lane, 7)
        zero16 = jnp.zeros(16, jnp.int32)

        def fill_zer(k, _):
            plsc.store_scatter(zer, [2 * k + hi_half, lane8],
                               jnp.zeros(16, jnp.float32))
            return _

        lax.fori_loop(0, SLAB // 2, fill_zer, None)
        ones0 = rows8.at[0, 0]

        def fill_ones(k, _):
            plsc.store_scatter(ones0, [2 * k + hi_half, lane8],
                               jnp.full(16, 1.0, jnp.float32))
            return _

        lax.fori_loop(0, SUB // 2, fill_ones, None)
        pltpu.sync_copy(nl16, nlb)
        nl = jnp.max(nlb[...])

        def zero_agg(j, _):
            pltpu.sync_copy(zer, agg_sp.at[pl.ds(s * A8 + j * SLAB, SLAB)])
            return _

        nfull = A8 // SLAB
        lax.fori_loop(0, nfull, zero_agg, None)
        tail = A8 - nfull * SLAB
        if tail:
            pltpu.sync_copy(zer.at[pl.ds(0, tail)],
                            agg_sp.at[pl.ds(s * A8 + nfull * SLAB, tail)])

        @pl.when(s == 0)
        def _():
            pltpu.sync_copy(zer.at[pl.ds(0, DUM)], ys_sp.at[pl.ds(NP, DUM)])

        plsc.subcore_barrier()

        def edge_sweep(do_gather):
            def fire_idx(ch, b):
                rb = s * ERB + ch * NJ
                if do_gather:
                    pltpu.async_copy(src2d.at[pl.ds(rb, NJ)], isrc.at[b], isem)
                pltpu.async_copy(dst2d.at[pl.ds(rb, NJ)], idst.at[b], isem)

            def drain_scatters(b):
                for j in range(NJ):
                    src = rows8.at[b, j] if do_gather else ones0
                    pltpu.make_async_copy(src, agg_sp.at[idst.at[b, j]],
                                          ssem).wait()

            def do_chunk(ch, b):
                rb = s * ERB + ch * NJ
                if do_gather:
                    pltpu.make_async_copy(src2d.at[pl.ds(rb, NJ)], isrc.at[b],
                                          isem).wait()
                pltpu.make_async_copy(dst2d.at[pl.ds(rb, NJ)], idst.at[b],
                                      isem).wait()
                gds = []
                if do_gather:
                    gds = [
                        pltpu.async_copy(ys_sp.at[isrc.at[b, j]],
                                         rows8.at[b, j], gsem)
                        for j in range(NJ)
                    ]

                @pl.when(ch >= DEPTH)
                def _():
                    drain_scatters((b + DEPTH) % NSLOT)

                @pl.when(ch + DEPTH < NCH)
                def _():
                    fire_idx(ch + DEPTH, (b + DEPTH) % NSLOT)

                for j in range(NJ):
                    if do_gather:
                        gds[j].wait()
                        pltpu.async_copy(rows8.at[b, j],
                                         agg_sp.at[idst.at[b, j]], ssem,
                                         add=True)
                    else:
                        pltpu.async_copy(ones0, agg_sp.at[idst.at[b, j]],
                                         ssem, add=True)

            for ch0 in range(DEPTH):
                fire_idx(ch0, ch0)

            def quad(q, _):
                for p in range(NSLOT):
                    do_chunk(q * NSLOT + p, p)
                return _

            lax.fori_loop(0, NCH // NSLOT, quad, None)
            for ch0 in range(NCH - DEPTH, NCH):
                drain_scatters(ch0 % NSLOT)

        edge_sweep(do_gather=False)
        plsc.subcore_barrier()

        def setup_slab(j, _):
            roff = s * CHUNK + j * SLAB
            pltpu.sync_copy(agg_sp.at[pl.ds(roff, SLAB)], abuf)
            pltpu.sync_copy(zer.at[pl.ds(0, SLAB)],
                            agg_sp.at[pl.ds(roff, SLAB)])
            pltpu.sync_copy(lab2d.at[c, pl.ds(roff, SLAB)], lbuf.at[0])

            def dbody(k, _):
                d = plsc.load_gather(abuf, [k * 16 + lane, zero16])
                d = jnp.maximum(d, jnp.float32(1.0))
                dbuf[pl.ds(j * SLAB + k * 16, 16)] = _vrsqrt(d)
                return _

            lax.fori_loop(0, (SLAB + 15) // 16, dbody, None)

            def ybody(k, _):
                idxr = 2 * k + hi_half
                ne = plsc.load_gather(dbuf, [j * SLAB + idxr])
                l = plsc.load_gather(lbuf.at[0], [idxr, lane8])
                plsc.store_scatter(lbuf.at[0], [idxr, lane8], l * ne)
                return _

            lax.fori_loop(0, SLAB // 2, ybody, None)
            pltpu.sync_copy(lbuf.at[0], ys_sp.at[pl.ds(roff, SLAB)])
            return _

        lax.fori_loop(0, NSLAB, setup_slab, None)
        plsc.subcore_barrier()

        def iteration(it, _):
            edge_sweep(do_gather=True)
            plsc.subcore_barrier()

            def lab_fire(j, q):
                pltpu.async_copy(
                    lab2d.at[c, pl.ds(s * CHUNK + j * SLAB, SLAB)],
                    lbuf.at[q], lsem)

            def node_slab(j, q):
                roff = s * CHUNK + j * SLAB

                @pl.when(j + 1 < NSLAB)
                def _():
                    lab_fire(j + 1, q ^ 1)

                pltpu.sync_copy(agg_sp.at[pl.ds(roff, SLAB)], abuf)
                pltpu.sync_copy(zer.at[pl.ds(0, SLAB)],
                                agg_sp.at[pl.ds(roff, SLAB)])
                pltpu.make_async_copy(
                    lab2d.at[c, pl.ds(roff, SLAB)], lbuf.at[q], lsem).wait()

                def body(k, _):
                    idxr = 2 * k + hi_half
                    ne = plsc.load_gather(dbuf, [j * SLAB + idxr])
                    a = plsc.load_gather(abuf, [idxr, lane8])
                    l = plsc.load_gather(lbuf.at[q], [idxr, lane8])
                    y = ONE_MINUS_ALPHA * l + ALPHA * ne * a
                    y = jnp.minimum(jnp.maximum(y, jnp.float32(0.0)),
                                    jnp.float32(1.0))
                    plsc.store_scatter(abuf, [idxr, lane8], y)
                    plsc.store_scatter(lbuf.at[q], [idxr, lane8], y * ne)
                    return _

                lax.fori_loop(0, SLAB // 2, body, None)
                pltpu.sync_copy(lbuf.at[q], ys_sp.at[pl.ds(roff, SLAB)])

                @pl.when(it == nl - 1)
                def _():
                    pltpu.sync_copy(abuf, out_f.at[c, pl.ds(roff, SLAB)])

                return None

            lab_fire(0, 0)

            def node_pair(d, _):
                node_slab(2 * d, 0)
                node_slab(2 * d + 1, 1)
                return _

            lax.fori_loop(0, NSLAB // 2, node_pair, None)
            plsc.subcore_barrier()
            return _

        lax.fori_loop(0, nl, iteration, None)

    return prop, NP, EPAD


def kernel(labels, edge_index, num_layers):
    N, C = labels.shape
    E = edge_index.shape[1]
    prop, NP, EPAD = _build(N, E)
    src = edge_index[0].astype(jnp.int32)
    dst = edge_index[1].astype(jnp.int32)
    pad = NP + jnp.arange(EPAD - E, dtype=jnp.int32) % DUM
    src2d = jnp.concatenate([src, pad]).reshape(EPAD // SUB, SUB)
    dst2d = jnp.concatenate([dst, pad]).reshape(EPAD // SUB, SUB)
    labp = jnp.pad(labels, ((0, NP - N), (0, 0)))
    lab2d = jnp.stack([labp[:, :H], labp[:, H:]])
    nl16 = jnp.full((16,), num_layers, jnp.int32)
    out = prop(lab2d, src2d, dst2d, nl16)
    return jnp.concatenate([out[0, :N], out[1, :N]], axis=1)


# speedup vs baseline: 2.5189x; 1.1727x over previous
import functools

import jax
import jax.numpy as jnp
from jax import lax
from jax.experimental import pallas as pl
from jax.experimental.pallas import tpu as pltpu
from jax.experimental.pallas import tpu_sc as plsc

ALPHA = 0.9
ONE_MINUS_ALPHA = 0.1
NSUB = 16
NCORE = 2
H = 8
DUM = 128
KE = 512
SUB = 512
NJ = KE // SUB
NSLOT = 2
DEPTH = NSLOT // 2
NSLAB = 16


def _vrsqrt(v):
    i = plsc.bitcast(v, jnp.int32)
    h = jnp.int32(0x5F3759DF) - (i >> 1)
    r = plsc.bitcast(h, jnp.float32)
    for _ in range(3):
        r = r * (jnp.float32(1.5) - jnp.float32(0.5) * v * r * r)
    return r


def _build(N, E):
    CHUNK = ((N + NSUB - 1) // NSUB + 63) // 64 * 64
    NP = NSUB * CHUNK
    NSP = NP + DUM
    SLAB = CHUNK // NSLAB
    EPAD = (E + NSUB * KE * NSLOT - 1) // (NSUB * KE * NSLOT) * (NSUB * KE * NSLOT)
    ET = EPAD // NSUB
    NCH = ET // KE
    ERB = ET // SUB
    A8 = NSP // NSUB

    mesh = plsc.VectorSubcoreMesh(core_axis_name="c", subcore_axis_name="s")

    @functools.partial(
        pl.kernel,
        out_type=jax.ShapeDtypeStruct((NCORE, NP, H), jnp.float32),
        mesh=mesh,
        scratch_types=dict(
            ys_sp=pltpu.VMEM_SHARED((NSP, H), jnp.float32),
            agg_sp=pltpu.VMEM_SHARED((NSP, H), jnp.float32),
            zer=pltpu.VMEM((SLAB, H), jnp.float32),
            dbuf=pltpu.VMEM((CHUNK + 16,), jnp.float32),
            abuf=pltpu.VMEM((SLAB, H), jnp.float32),
            lbuf=pltpu.VMEM((2, SLAB, H), jnp.float32),
            isrc=pltpu.VMEM((NSLOT, NJ, SUB), jnp.int32),
            idst=pltpu.VMEM((NSLOT, NJ, SUB), jnp.int32),
            rows8=pltpu.VMEM((NSLOT, NJ, SUB, H), jnp.float32),
            nlb=pltpu.VMEM((16,), jnp.int32),
            gsem=pltpu.SemaphoreType.DMA,
            lsem=pltpu.SemaphoreType.DMA,
            isem=pltpu.SemaphoreType.DMA,
            ssem=pltpu.SemaphoreType.DMA,
        ),
        compiler_params=pltpu.CompilerParams(needs_layout_passes=False,
                                             use_tc_tiling_on_sc=False),
    )
    def prop(lab2d, src2d, dst2d, nl16, out_f, ys_sp, agg_sp, zer, dbuf,
             abuf, lbuf, isrc, idst, rows8, nlb, gsem, lsem, isem, ssem):
        c = lax.axis_index("c")
        s = lax.axis_index("s")
        lane = lax.iota(jnp.int32, 16)
        hi_half = lane >> 3
        lane8 = jnp.bitwise_and(lane, 7)
        zero16 = jnp.zeros(16, jnp.int32)

        def fill_zer(k, _):
            plsc.store_scatter(zer, [2 * k + hi_half, lane8],
                               jnp.zeros(16, jnp.float32))
            return _

        lax.fori_loop(0, SLAB // 2, fill_zer, None)
        ones0 = rows8.at[0, 0]

        def fill_ones(k, _):
            plsc.store_scatter(ones0, [2 * k + hi_half, lane8],
                               jnp.full(16, 1.0, jnp.float32))
            return _

        lax.fori_loop(0, SUB // 2, fill_ones, None)
        pltpu.sync_copy(nl16, nlb)
        nl = jnp.max(nlb[...])

        def zero_agg(j, _):
            pltpu.sync_copy(zer, agg_sp.at[pl.ds(s * A8 + j * SLAB, SLAB)])
            return _

        nfull = A8 // SLAB
        lax.fori_loop(0, nfull, zero_agg, None)
        tail = A8 - nfull * SLAB
        if tail:
            pltpu.sync_copy(zer.at[pl.ds(0, tail)],
                            agg_sp.at[pl.ds(s * A8 + nfull * SLAB, tail)])

        @pl.when(s == 0)
        def _():
            pltpu.sync_copy(zer.at[pl.ds(0, DUM)], ys_sp.at[pl.ds(NP, DUM)])

        plsc.subcore_barrier()

        def edge_sweep(do_gather):
            def fire_idx(ch, b):
                rb = s * ERB + ch * NJ
                if do_gather:
                    pltpu.async_copy(src2d.at[pl.ds(rb, NJ)], isrc.at[b], isem)
                pltpu.async_copy(dst2d.at[pl.ds(rb, NJ)], idst.at[b], isem)

            def drain_scatters(b):
                for j in range(NJ):
                    src = rows8.at[b, j] if do_gather else ones0
                    pltpu.make_async_copy(src, agg_sp.at[idst.at[b, j]],
                                          ssem).wait()

            def do_chunk(ch, b):
                rb = s * ERB + ch * NJ
                if do_gather:
                    pltpu.make_async_copy(src2d.at[pl.ds(rb, NJ)], isrc.at[b],
                                          isem).wait()
                pltpu.make_async_copy(dst2d.at[pl.ds(rb, NJ)], idst.at[b],
                                      isem).wait()
                gds = []
                if do_gather:
                    gds = [
                        pltpu.async_copy(ys_sp.at[isrc.at[b, j]],
                                         rows8.at[b, j], gsem)
                        for j in range(NJ)
                    ]

                @pl.when(ch >= DEPTH)
                def _():
                    drain_scatters((b + DEPTH) % NSLOT)

                @pl.when(ch + DEPTH < NCH)
                def _():
                    fire_idx(ch + DEPTH, (b + DEPTH) % NSLOT)

                for j in range(NJ):
                    if do_gather:
                        gds[j].wait()
                        pltpu.async_copy(rows8.at[b, j],
                                         agg_sp.at[idst.at[b, j]], ssem,
                                         add=True)
                    else:
                        pltpu.async_copy(ones0, agg_sp.at[idst.at[b, j]],
                                         ssem, add=True)

            for ch0 in range(DEPTH):
                fire_idx(ch0, ch0)

            def quad(q, _):
                for p in range(NSLOT):
                    do_chunk(q * NSLOT + p, p)
                return _

            lax.fori_loop(0, NCH // NSLOT, quad, None)
            for ch0 in range(NCH - DEPTH, NCH):
                drain_scatters(ch0 % NSLOT)

        edge_sweep(do_gather=False)
        plsc.subcore_barrier()

        def setup_slab(j, _):
            roff = s * CHUNK + j * SLAB
            pltpu.sync_copy(agg_sp.at[pl.ds(roff, SLAB)], abuf)
            pltpu.sync_copy(zer.at[pl.ds(0, SLAB)],
                            agg_sp.at[pl.ds(roff, SLAB)])
            pltpu.sync_copy(lab2d.at[c, pl.ds(roff, SLAB)], lbuf.at[0])

            def dbody(k, _):
                d = plsc.load_gather(abuf, [k * 16 + lane, zero16])
                d = jnp.maximum(d, jnp.float32(1.0))
                dbuf[pl.ds(j * SLAB + k * 16, 16)] = _vrsqrt(d)
                return _

            lax.fori_loop(0, (SLAB + 15) // 16, dbody, None)

            def ybody(k, _):
                idxr = 2 * k + hi_half
                ne = plsc.load_gather(dbuf, [j * SLAB + idxr])
                l = plsc.load_gather(lbuf.at[0], [idxr, lane8])
                plsc.store_scatter(lbuf.at[0], [idxr, lane8], l * ne)
                return _

            lax.fori_loop(0, SLAB // 2, ybody, None)
            pltpu.sync_copy(lbuf.at[0], ys_sp.at[pl.ds(roff, SLAB)])
            return _

        lax.fori_loop(0, NSLAB, setup_slab, None)
        plsc.subcore_barrier()

        def iteration(it, _):
            edge_sweep(do_gather=True)
            plsc.subcore_barrier()

            def lab_fire(j, q):
                pltpu.async_copy(
                    lab2d.at[c, pl.ds(s * CHUNK + j * SLAB, SLAB)],
                    lbuf.at[q], lsem)

            def node_slab(j, q):
                roff = s * CHUNK + j * SLAB

                @pl.when(j + 1 < NSLAB)
                def _():
                    lab_fire(j + 1, q ^ 1)

                pltpu.sync_copy(agg_sp.at[pl.ds(roff, SLAB)], abuf)
                pltpu.sync_copy(zer.at[pl.ds(0, SLAB)],
                                agg_sp.at[pl.ds(roff, SLAB)])
                pltpu.make_async_copy(
                    lab2d.at[c, pl.ds(roff, SLAB)], lbuf.at[q], lsem).wait()

                @functools.partial(plsc.parallel_loop, 0, SLAB // 2,
                                   unroll=4)
                def _(k):
                    idxr = 2 * k + hi_half
                    ne = plsc.load_gather(dbuf, [j * SLAB + idxr])
                    a = plsc.load_gather(abuf, [idxr, lane8])
                    l = plsc.load_gather(lbuf.at[q], [idxr, lane8])
                    y = ONE_MINUS_ALPHA * l + ALPHA * ne * a
                    y = jnp.minimum(jnp.maximum(y, jnp.float32(0.0)),
                                    jnp.float32(1.0))
                    plsc.store_scatter(abuf, [idxr, lane8], y)
                    plsc.store_scatter(lbuf.at[q], [idxr, lane8], y * ne)
                pltpu.sync_copy(lbuf.at[q], ys_sp.at[pl.ds(roff, SLAB)])

                @pl.when(it == nl - 1)
                def _():
                    pltpu.sync_copy(abuf, out_f.at[c, pl.ds(roff, SLAB)])

                return None

            lab_fire(0, 0)

            def node_pair(d, _):
                node_slab(2 * d, 0)
                node_slab(2 * d + 1, 1)
                return _

            lax.fori_loop(0, NSLAB // 2, node_pair, None)
            plsc.subcore_barrier()
            return _

        lax.fori_loop(0, nl, iteration, None)

    return prop, NP, EPAD


def kernel(labels, edge_index, num_layers):
    N, C = labels.shape
    E = edge_index.shape[1]
    prop, NP, EPAD = _build(N, E)
    src = edge_index[0].astype(jnp.int32)
    dst = edge_index[1].astype(jnp.int32)
    pad = NP + jnp.arange(EPAD - E, dtype=jnp.int32) % DUM
    src2d = jnp.concatenate([src, pad]).reshape(EPAD // SUB, SUB)
    dst2d = jnp.concatenate([dst, pad]).reshape(EPAD // SUB, SUB)
    labp = jnp.pad(labels, ((0, NP - N), (0, 0)))
    lab2d = jnp.stack([labp[:, :H], labp[:, H:]])
    nl16 = jnp.full((16,), num_layers, jnp.int32)
    out = prop(lab2d, src2d, dst2d, nl16)
    return jnp.concatenate([out[0, :N], out[1, :N]], axis=1)
